# Initial kernel scaffold; baseline (speedup 1.0000x reference)
import jax
import jax.numpy as jnp
from jax.experimental import pallas as pl


def kernel(x, edge_index, edge_attr, batch, depth, W_i, W_m, W_a, ffn_w1, ffn_b1, ffn_w2, ffn_b2, last_w, last_b):
    raise NotImplementedError("write your pallas kernel here")


# speedup vs baseline: 6.8380x; 6.8380x over previous
import jax
import jax.numpy as jnp
from jax import lax
from jax.experimental import pallas as pl
from jax.experimental.pallas import tpu as pltpu
from jax.experimental.pallas import tpu_sc as plsc

N = 10000
E = 320000
ATOM = 4
BOND = 3
HID = 128
G = 128
OUT = 16
DEPTH = 3

NC = 2
NS = 16
NW = NC * NS
NPAD = 10240
CHUNK = 80
IBLK = 25
NBLK = 5
EPW = E // NW
RPS = NPAD // NS
ZROWS = 8
H0_CHUNKS = N // CHUNK

BR = 2048
NB = NPAD // BR

_F32 = jnp.float32
_MM = dict(preferred_element_type=jnp.float32,
           precision=jax.lax.Precision.HIGHEST)

assert NW * NBLK * IBLK * CHUNK == E
assert EPW == NBLK * IBLK * CHUNK


def _dot(a, b):
    return jax.lax.dot_general(a, b, (((1,), (0,)), ((), ())), **_MM)


def _mm_body(x_ref, w_ref, o_ref):
    o_ref[...] = _dot(x_ref[...], w_ref[...])


def _tc_matmul(x, w, br):
    rows, k = x.shape
    return pl.pallas_call(
        _mm_body,
        grid=(rows // br,),
        in_specs=[
            pl.BlockSpec((br, k), lambda i: (i, 0)),
            pl.BlockSpec(w.shape, lambda i: (0, 0)),
        ],
        out_specs=pl.BlockSpec((br, w.shape[1]), lambda i: (i, 0)),
        out_shape=jax.ShapeDtypeStruct((rows, w.shape[1]), _F32),
    )(x, w)


def _p_body(h_ref, w_ref, p_ref):
    p_ref[...] = jnp.maximum(_dot(h_ref[...], w_ref[...]), 0.0)


def _tc_p(h, w_m):
    return pl.pallas_call(
        _p_body,
        grid=(NB,),
        in_specs=[
            pl.BlockSpec((BR, HID), lambda i: (i, 0)),
            pl.BlockSpec((HID, HID), lambda i: (0, 0)),
        ],
        out_specs=pl.BlockSpec((BR, HID), lambda i: (i, 0)),
        out_shape=jax.ShapeDtypeStruct((NPAD, HID), _F32),
    )(h, w_m)


def _prop_body(h_ref, a0_ref, a1_ref, w_ref, hn_ref, p_ref):
    hn = h_ref[...] + a0_ref[...] + a1_ref[...]
    hn_ref[...] = hn
    p_ref[...] = jnp.maximum(_dot(hn, w_ref[...]), 0.0)


def _tc_prop(h, agg, w_m):
    return pl.pallas_call(
        _prop_body,
        grid=(NB,),
        in_specs=[
            pl.BlockSpec((BR, HID), lambda i: (i, 0)),
            pl.BlockSpec((BR, HID), lambda i: (i, 0)),
            pl.BlockSpec((BR, HID), lambda i: (i + NB, 0)),
            pl.BlockSpec((HID, HID), lambda i: (0, 0)),
        ],
        out_specs=[
            pl.BlockSpec((BR, HID), lambda i: (i, 0)),
            pl.BlockSpec((BR, HID), lambda i: (i, 0)),
        ],
        out_shape=[
            jax.ShapeDtypeStruct((NPAD, HID), _F32),
            jax.ShapeDtypeStruct((NPAD, HID), _F32),
        ],
    )(h, agg, agg, w_m)


def _readout_body(x_ref, n0_ref, n1_ref, batch_ref, wa1_ref, wa2_ref,
                  fw1_ref, fb1_ref, fw2_ref, fb2_ref, lw_ref, lb_ref,
                  o_ref, pooled_ref):
    i = pl.program_id(0)
    node = n0_ref[...] + n1_ref[...]
    emb = jnp.maximum(_dot(x_ref[...], wa1_ref[...])
                      + _dot(node, wa2_ref[...]), 0.0)
    onehot = (batch_ref[...] ==
              jax.lax.broadcasted_iota(jnp.int32, (BR, G), 1)).astype(_F32)
    contrib = jax.lax.dot_general(onehot, emb, (((0,), (0,)), ((), ())), **_MM)

    @pl.when(i == 0)
    def _():
        pooled_ref[...] = jnp.zeros_like(pooled_ref)

    pooled_ref[...] += contrib

    @pl.when(i == pl.num_programs(0) - 1)
    def _():
        g1 = jnp.maximum(_dot(pooled_ref[...], fw1_ref[...]) + fb1_ref[...],
                         0.0)
        g2 = _dot(g1, fw2_ref[...]) + fb2_ref[...]
        o_ref[...] = _dot(g2, lw_ref[...]) + lb_ref[...]


def _tc_readout(xp, nacc, batch2d, wa1, wa2, fw1, fb1, fw2, fb2, lw, lb):
    full = lambda a: pl.BlockSpec(a.shape, lambda i: tuple(0 for _ in a.shape))
    return pl.pallas_call(
        _readout_body,
        grid=(NB,),
        in_specs=[
            pl.BlockSpec((BR, ATOM), lambda i: (i, 0)),
            pl.BlockSpec((BR, HID), lambda i: (i, 0)),
            pl.BlockSpec((BR, HID), lambda i: (i + NB, 0)),
            pl.BlockSpec((BR, 1), lambda i: (i, 0)),
            full(wa1), full(wa2), full(fw1), full(fb1), full(fw2), full(fb2),
            full(lw), full(lb),
        ],
        out_specs=pl.BlockSpec((G, OUT), lambda i: (0, 0)),
        out_shape=jax.ShapeDtypeStruct((G, OUT), _F32),
        scratch_shapes=[pltpu.VMEM((G, HID), _F32)],
    )(xp, nacc, nacc, batch2d, wa1, wa2, fw1, fb1, fw2, fb2, lw, lb)


_MESH = plsc.VectorSubcoreMesh(core_axis_name="c", subcore_axis_name="s")


def _ids():
    c = lax.axis_index("c")
    t = lax.axis_index("s")
    return c, t, t * NC + c


def _fill_zero(zero_v):
    def zrow(i, _):
        for j in range(HID // 16):
            zero_v[i, pl.ds(j * 16, 16)] = jnp.zeros((16,), _F32)
        return 0
    lax.fori_loop(0, ZROWS, zrow, 0, unroll=False)


def _zero_shared(zero_v, acc_sh):
    t = lax.axis_index("s")
    for z in range(RPS // ZROWS):
        pltpu.sync_copy(zero_v, acc_sh.at[pl.ds(t * RPS + z * ZROWS, ZROWS)])


def _flush_shared(acc_sh, out_hbm):
    c = lax.axis_index("c")
    t = lax.axis_index("s")
    pltpu.sync_copy(acc_sh.at[pl.ds(t * RPS, RPS)],
                    out_hbm.at[pl.ds(c * NPAD + t * RPS, RPS)])


def _relu_add(qrows, rrows):
    def row(i, _):
        for j in range(HID // 16):
            sl = pl.ds(j * 16, 16)
            qrows[i, sl] = jnp.maximum(qrows[i, sl] + rrows[i, sl], 0.0)
        return 0
    lax.fori_loop(0, CHUNK, row, 0, unroll=False)


def _sc_h0_body(q_hbm, r_hbm, src_hbm, h0_hbm, src_v, qrows, rrows, sem):
    _, _, w = _ids()
    for g in range(4):
        ch = w * 4 + g

        @pl.when(ch < H0_CHUNKS)
        def _():
            pltpu.sync_copy(src_hbm.at[ch], src_v)
            pltpu.async_copy(q_hbm.at[src_v.at[0]], qrows, sem).wait()
            pltpu.sync_copy(r_hbm.at[pl.ds(ch * CHUNK, CHUNK)], rrows)
            _relu_add(qrows, rrows)
            pltpu.sync_copy(qrows, h0_hbm.at[pl.ds(ch * CHUNK, CHUNK)])


def _sc_h0(q, r, srcA3):
    kern = pl.kernel(
        _sc_h0_body,
        out_type=jax.ShapeDtypeStruct((NPAD, HID), _F32),
        mesh=_MESH,
        scratch_types=[
            pltpu.VMEM((1, CHUNK), jnp.int32),
            pltpu.VMEM((CHUNK, HID), _F32),
            pltpu.VMEM((CHUNK, HID), _F32),
            pltpu.SemaphoreType.DMA,
        ],
    )
    return kern(q, r, srcA3)


def _sc_spmm_body(p_hbm, src_hbm, dst_hbm, out_hbm,
                  src_v, dst_v, rows_v, zero_v, agg_sh, sem):
    _, _, w = _ids()
    _fill_zero(zero_v)
    _zero_shared(zero_v, agg_sh)
    plsc.subcore_barrier()
    for b in range(NBLK):
        pltpu.sync_copy(src_hbm.at[w, b], src_v)
        pltpu.sync_copy(dst_hbm.at[w, b], dst_v)

        def body(g, _):
            pltpu.async_copy(p_hbm.at[src_v.at[g]], rows_v, sem).wait()
            pltpu.sync_copy(rows_v, agg_sh.at[dst_v.at[g]], add=True)
            return 0

        lax.fori_loop(0, IBLK, body, 0, unroll=False)
    plsc.subcore_barrier()
    _flush_shared(agg_sh, out_hbm)


def _sc_spmm(p, src4, dst4):
    kern = pl.kernel(
        _sc_spmm_body,
        out_type=jax.ShapeDtypeStruct((NC * NPAD, HID), _F32),
        mesh=_MESH,
        scratch_types=[
            pltpu.VMEM((IBLK, CHUNK), jnp.int32),
            pltpu.VMEM((IBLK, CHUNK), jnp.int32),
            pltpu.VMEM((CHUNK, HID), _F32),
            pltpu.VMEM((ZROWS, HID), _F32),
            pltpu.VMEM_SHARED((NPAD, HID), _F32),
            pltpu.SemaphoreType.DMA,
        ],
    )
    return kern(p, src4, dst4)


def _sc_node_body(q_hbm, r_hbm, h3_hbm, src_hbm, dst_hbm, out_hbm,
                  src_v, dst_v, qrows, rrows, zero_v, acc_sh, sem):
    _, _, w = _ids()
    _fill_zero(zero_v)
    _zero_shared(zero_v, acc_sh)
    plsc.subcore_barrier()
    for b in range(NBLK):
        pltpu.sync_copy(src_hbm.at[w, b], src_v)
        pltpu.sync_copy(dst_hbm.at[w, b], dst_v)

        def body(g, _):
            @pl.when(w == 0)
            def _():
                pltpu.sync_copy(
                    h3_hbm.at[pl.ds((b * IBLK + g) * CHUNK, CHUNK)], qrows)

            @pl.when(w != 0)
            def _():
                pltpu.async_copy(q_hbm.at[src_v.at[g]], qrows, sem).wait()
                pltpu.sync_copy(
                    r_hbm.at[pl.ds(w * EPW + (b * IBLK + g) * CHUNK, CHUNK)],
                    rrows)
                _relu_add(qrows, rrows)

            pltpu.sync_copy(qrows, acc_sh.at[dst_v.at[g]], add=True)
            return 0

        lax.fori_loop(0, IBLK, body, 0, unroll=False)
    plsc.subcore_barrier()
    _flush_shared(acc_sh, out_hbm)


def _sc_node(q, r, h3, src4, dst4):
    kern = pl.kernel(
        _sc_node_body,
        out_type=jax.ShapeDtypeStruct((NC * NPAD, HID), _F32),
        mesh=_MESH,
        scratch_types=[
            pltpu.VMEM((IBLK, CHUNK), jnp.int32),
            pltpu.VMEM((IBLK, CHUNK), jnp.int32),
            pltpu.VMEM((CHUNK, HID), _F32),
            pltpu.VMEM((CHUNK, HID), _F32),
            pltpu.VMEM((ZROWS, HID), _F32),
            pltpu.VMEM_SHARED((NPAD, HID), _F32),
            pltpu.SemaphoreType.DMA,
        ],
    )
    return kern(q, r, h3, src4, dst4)


def kernel(x, edge_index, edge_attr, batch, depth, W_i, W_m, W_a,
           ffn_w1, ffn_b1, ffn_w2, ffn_b2, last_w, last_b):
    src4 = edge_index[0].reshape(NW, NBLK, IBLK, CHUNK)
    dst4 = edge_index[1].reshape(NW, NBLK, IBLK, CHUNK)
    srcA3 = edge_index[0, :N].reshape(H0_CHUNKS, 1, CHUNK)
    xp = jnp.pad(x, ((0, NPAD - N), (0, 0)))
    batch2d = jnp.pad(batch, (0, NPAD - N),
                      constant_values=G).reshape(NPAD, 1)
    wi1, wi2 = W_i[:ATOM], W_i[ATOM:]
    wa1, wa2 = W_a[:ATOM], W_a[ATOM:]
    fb1 = ffn_b1.reshape(1, -1)
    fb2 = ffn_b2.reshape(1, -1)
    lb = last_b.reshape(1, -1)

    q = _tc_matmul(xp, wi1, BR)
    r = _tc_matmul(edge_attr, wi2, 8000)
    h = _sc_h0(q, r, srcA3)
    p = _tc_p(h, W_m)
    for _ in range(DEPTH):
        agg = _sc_spmm(p, src4, dst4)
        h, p = _tc_prop(h, agg, W_m)
    nacc = _sc_node(q, r, h, src4, dst4)
    return _tc_readout(xp, nacc, batch2d, wa1, wa2,
                       ffn_w1, fb1, ffn_w2, fb2, last_w, lb)


# speedup vs baseline: 8.3296x; 1.2181x over previous
import jax
import jax.numpy as jnp
from jax import lax
from jax.experimental import pallas as pl
from jax.experimental.pallas import tpu as pltpu
from jax.experimental.pallas import tpu_sc as plsc

N = 10000
E = 320000
ATOM = 4
BOND = 3
HID = 128
G = 128
OUT = 16
DEPTH = 3

NC = 2
NS = 16
NW = NC * NS
NPAD = 10240
CHUNK = 80
IBLK = 25
NBLK = 5
EPW = E // NW
RPS = NPAD // NS
ZROWS = 8
H0_CHUNKS = N // CHUNK

BR = 2048
NB = NPAD // BR

_F32 = jnp.float32
_MM = dict(preferred_element_type=jnp.float32,
           precision=jax.lax.Precision.HIGHEST)

assert NW * NBLK * IBLK * CHUNK == E
assert EPW == NBLK * IBLK * CHUNK


def _dot(a, b):
    return jax.lax.dot_general(a, b, (((1,), (0,)), ((), ())), **_MM)


def _mm_body(x_ref, w_ref, o_ref):
    o_ref[...] = _dot(x_ref[...], w_ref[...])


def _tc_matmul(x, w, br):
    rows, k = x.shape
    return pl.pallas_call(
        _mm_body,
        grid=(rows // br,),
        in_specs=[
            pl.BlockSpec((br, k), lambda i: (i, 0)),
            pl.BlockSpec(w.shape, lambda i: (0, 0)),
        ],
        out_specs=pl.BlockSpec((br, w.shape[1]), lambda i: (i, 0)),
        out_shape=jax.ShapeDtypeStruct((rows, w.shape[1]), _F32),
    )(x, w)


def _p_body(h_ref, w_ref, p_ref):
    p_ref[...] = jnp.maximum(_dot(h_ref[...], w_ref[...]), 0.0)


def _tc_p(h, w_m):
    return pl.pallas_call(
        _p_body,
        grid=(NB,),
        in_specs=[
            pl.BlockSpec((BR, HID), lambda i: (i, 0)),
            pl.BlockSpec((HID, HID), lambda i: (0, 0)),
        ],
        out_specs=pl.BlockSpec((BR, HID), lambda i: (i, 0)),
        out_shape=jax.ShapeDtypeStruct((NPAD, HID), _F32),
    )(h, w_m)


def _prop_body(h_ref, a0_ref, a1_ref, w_ref, hn_ref, p_ref):
    hn = h_ref[...] + a0_ref[...] + a1_ref[...]
    hn_ref[...] = hn
    p_ref[...] = jnp.maximum(_dot(hn, w_ref[...]), 0.0)


def _tc_prop(h, agg, w_m):
    return pl.pallas_call(
        _prop_body,
        grid=(NB,),
        in_specs=[
            pl.BlockSpec((BR, HID), lambda i: (i, 0)),
            pl.BlockSpec((BR, HID), lambda i: (i, 0)),
            pl.BlockSpec((BR, HID), lambda i: (i + NB, 0)),
            pl.BlockSpec((HID, HID), lambda i: (0, 0)),
        ],
        out_specs=[
            pl.BlockSpec((BR, HID), lambda i: (i, 0)),
            pl.BlockSpec((BR, HID), lambda i: (i, 0)),
        ],
        out_shape=[
            jax.ShapeDtypeStruct((NPAD, HID), _F32),
            jax.ShapeDtypeStruct((NPAD, HID), _F32),
        ],
    )(h, agg, agg, w_m)


def _readout_body(x_ref, n0_ref, n1_ref, batch_ref, wa1_ref, wa2_ref,
                  fw1_ref, fb1_ref, fw2_ref, fb2_ref, lw_ref, lb_ref,
                  o_ref, pooled_ref):
    i = pl.program_id(0)
    node = n0_ref[...] + n1_ref[...]
    emb = jnp.maximum(_dot(x_ref[...], wa1_ref[...])
                      + _dot(node, wa2_ref[...]), 0.0)
    onehot = (batch_ref[...] ==
              jax.lax.broadcasted_iota(jnp.int32, (BR, G), 1)).astype(_F32)
    contrib = jax.lax.dot_general(onehot, emb, (((0,), (0,)), ((), ())), **_MM)

    @pl.when(i == 0)
    def _():
        pooled_ref[...] = jnp.zeros_like(pooled_ref)

    pooled_ref[...] += contrib

    @pl.when(i == pl.num_programs(0) - 1)
    def _():
        g1 = jnp.maximum(_dot(pooled_ref[...], fw1_ref[...]) + fb1_ref[...],
                         0.0)
        g2 = _dot(g1, fw2_ref[...]) + fb2_ref[...]
        o_ref[...] = _dot(g2, lw_ref[...]) + lb_ref[...]


def _tc_readout(xp, nacc, batch2d, wa1, wa2, fw1, fb1, fw2, fb2, lw, lb):
    full = lambda a: pl.BlockSpec(a.shape, lambda i: tuple(0 for _ in a.shape))
    return pl.pallas_call(
        _readout_body,
        grid=(NB,),
        in_specs=[
            pl.BlockSpec((BR, ATOM), lambda i: (i, 0)),
            pl.BlockSpec((BR, HID), lambda i: (i, 0)),
            pl.BlockSpec((BR, HID), lambda i: (i + NB, 0)),
            pl.BlockSpec((BR, 1), lambda i: (i, 0)),
            full(wa1), full(wa2), full(fw1), full(fb1), full(fw2), full(fb2),
            full(lw), full(lb),
        ],
        out_specs=pl.BlockSpec((G, OUT), lambda i: (0, 0)),
        out_shape=jax.ShapeDtypeStruct((G, OUT), _F32),
        scratch_shapes=[pltpu.VMEM((G, HID), _F32)],
    )(xp, nacc, nacc, batch2d, wa1, wa2, fw1, fb1, fw2, fb2, lw, lb)


_MESH = plsc.VectorSubcoreMesh(core_axis_name="c", subcore_axis_name="s")


def _ids():
    c = lax.axis_index("c")
    t = lax.axis_index("s")
    return c, t, t * NC + c


def _fill_zero(zero_v):
    def zrow(i, _):
        for j in range(HID // 16):
            zero_v[i, pl.ds(j * 16, 16)] = jnp.zeros((16,), _F32)
        return 0
    lax.fori_loop(0, ZROWS, zrow, 0, unroll=False)


def _zero_shared(zero_v, acc_sh):
    t = lax.axis_index("s")
    for z in range(RPS // ZROWS):
        pltpu.sync_copy(zero_v, acc_sh.at[pl.ds(t * RPS + z * ZROWS, ZROWS)])


def _flush_shared(acc_sh, out_hbm):
    c = lax.axis_index("c")
    t = lax.axis_index("s")
    pltpu.sync_copy(acc_sh.at[pl.ds(t * RPS, RPS)],
                    out_hbm.at[pl.ds(c * NPAD + t * RPS, RPS)])


def _relu_add(qrows, rrows):
    def row(i, _):
        for j in range(HID // 16):
            sl = pl.ds(j * 16, 16)
            qrows[i, sl] = jnp.maximum(qrows[i, sl] + rrows[i, sl], 0.0)
        return 0
    lax.fori_loop(0, CHUNK, row, 0, unroll=False)


def _sc_h0_body(q_hbm, r_hbm, src_hbm, h0_hbm, src_v, qrows, rrows, sem):
    _, _, w = _ids()
    for g in range(4):
        ch = w * 4 + g

        @pl.when(ch < H0_CHUNKS)
        def _():
            pltpu.sync_copy(src_hbm.at[ch], src_v)
            pltpu.async_copy(q_hbm.at[src_v.at[0]], qrows, sem).wait()
            pltpu.sync_copy(r_hbm.at[pl.ds(ch * CHUNK, CHUNK)], rrows)
            _relu_add(qrows, rrows)
            pltpu.sync_copy(qrows, h0_hbm.at[pl.ds(ch * CHUNK, CHUNK)])


def _sc_h0(q, r, srcA3):
    kern = pl.kernel(
        _sc_h0_body,
        out_type=jax.ShapeDtypeStruct((NPAD, HID), _F32),
        mesh=_MESH,
        scratch_types=[
            pltpu.VMEM((1, CHUNK), jnp.int32),
            pltpu.VMEM((CHUNK, HID), _F32),
            pltpu.VMEM((CHUNK, HID), _F32),
            pltpu.SemaphoreType.DMA,
        ],
    )
    return kern(q, r, srcA3)


def _pipelined_block(p_hbm, src_v, dst_v, rows0, rows1, agg_sh, sem0, sem1):
    pltpu.async_copy(p_hbm.at[src_v.at[0]], rows0, sem0)

    def body(g, _):
        even = lax.rem(g, 2) == 0

        @pl.when(even)
        def _():
            @pl.when(g + 1 < IBLK)
            def _():
                pltpu.async_copy(p_hbm.at[src_v.at[g + 1]], rows1, sem1)
            pltpu.make_async_copy(p_hbm.at[src_v.at[g]], rows0, sem0).wait()
            pltpu.sync_copy(rows0, agg_sh.at[dst_v.at[g]], add=True)

        @pl.when(jnp.logical_not(even))
        def _():
            @pl.when(g + 1 < IBLK)
            def _():
                pltpu.async_copy(p_hbm.at[src_v.at[g + 1]], rows0, sem0)
            pltpu.make_async_copy(p_hbm.at[src_v.at[g]], rows1, sem1).wait()
            pltpu.sync_copy(rows1, agg_sh.at[dst_v.at[g]], add=True)

        return 0

    lax.fori_loop(0, IBLK, body, 0, unroll=False)


def _sc_spmm_body(p_hbm, src_hbm, dst_hbm, out_hbm,
                  src_v, dst_v, rows0, rows1, zero_v, agg_sh, sem0, sem1):
    _, _, w = _ids()
    _fill_zero(zero_v)
    _zero_shared(zero_v, agg_sh)
    plsc.subcore_barrier()
    for b in range(NBLK):
        pltpu.sync_copy(src_hbm.at[w, b], src_v)
        pltpu.sync_copy(dst_hbm.at[w, b], dst_v)
        _pipelined_block(p_hbm, src_v, dst_v, rows0, rows1, agg_sh,
                         sem0, sem1)
    plsc.subcore_barrier()
    _flush_shared(agg_sh, out_hbm)


def _sc_spmm(p, src4, dst4):
    kern = pl.kernel(
        _sc_spmm_body,
        out_type=jax.ShapeDtypeStruct((NC * NPAD, HID), _F32),
        mesh=_MESH,
        scratch_types=[
            pltpu.VMEM((IBLK, CHUNK), jnp.int32),
            pltpu.VMEM((IBLK, CHUNK), jnp.int32),
            pltpu.VMEM((CHUNK, HID), _F32),
            pltpu.VMEM((CHUNK, HID), _F32),
            pltpu.VMEM((ZROWS, HID), _F32),
            pltpu.VMEM_SHARED((NPAD, HID), _F32),
            pltpu.SemaphoreType.DMA,
            pltpu.SemaphoreType.DMA,
        ],
    )
    return kern(p, src4, dst4)


def _sc_node_body(q_hbm, r_hbm, h3_hbm, src_hbm, dst_hbm, out_hbm,
                  src_v, dst_v, qrows, rrows, zero_v, acc_sh, sem):
    _, _, w = _ids()
    _fill_zero(zero_v)
    _zero_shared(zero_v, acc_sh)
    plsc.subcore_barrier()
    for b in range(NBLK):
        pltpu.sync_copy(src_hbm.at[w, b], src_v)
        pltpu.sync_copy(dst_hbm.at[w, b], dst_v)

        def body(g, _):
            @pl.when(w == 0)
            def _():
                pltpu.sync_copy(
                    h3_hbm.at[pl.ds((b * IBLK + g) * CHUNK, CHUNK)], qrows)

            @pl.when(w != 0)
            def _():
                pltpu.async_copy(q_hbm.at[src_v.at[g]], qrows, sem).wait()
                pltpu.sync_copy(
                    r_hbm.at[pl.ds(w * EPW + (b * IBLK + g) * CHUNK, CHUNK)],
                    rrows)
                _relu_add(qrows, rrows)

            pltpu.sync_copy(qrows, acc_sh.at[dst_v.at[g]], add=True)
            return 0

        lax.fori_loop(0, IBLK, body, 0, unroll=False)
    plsc.subcore_barrier()
    _flush_shared(acc_sh, out_hbm)


def _sc_node(q, r, h3, src4, dst4):
    kern = pl.kernel(
        _sc_node_body,
        out_type=jax.ShapeDtypeStruct((NC * NPAD, HID), _F32),
        mesh=_MESH,
        scratch_types=[
            pltpu.VMEM((IBLK, CHUNK), jnp.int32),
            pltpu.VMEM((IBLK, CHUNK), jnp.int32),
            pltpu.VMEM((CHUNK, HID), _F32),
            pltpu.VMEM((CHUNK, HID), _F32),
            pltpu.VMEM((ZROWS, HID), _F32),
            pltpu.VMEM_SHARED((NPAD, HID), _F32),
            pltpu.SemaphoreType.DMA,
        ],
    )
    return kern(q, r, h3, src4, dst4)


def kernel(x, edge_index, edge_attr, batch, depth, W_i, W_m, W_a,
           ffn_w1, ffn_b1, ffn_w2, ffn_b2, last_w, last_b):
    src4 = edge_index[0].reshape(NW, NBLK, IBLK, CHUNK)
    dst4 = edge_index[1].reshape(NW, NBLK, IBLK, CHUNK)
    srcA3 = edge_index[0, :N].reshape(H0_CHUNKS, 1, CHUNK)
    xp = jnp.pad(x, ((0, NPAD - N), (0, 0)))
    batch2d = jnp.pad(batch, (0, NPAD - N),
                      constant_values=G).reshape(NPAD, 1)
    wi1, wi2 = W_i[:ATOM], W_i[ATOM:]
    wa1, wa2 = W_a[:ATOM], W_a[ATOM:]
    fb1 = ffn_b1.reshape(1, -1)
    fb2 = ffn_b2.reshape(1, -1)
    lb = last_b.reshape(1, -1)

    q = _tc_matmul(xp, wi1, BR)
    r = _tc_matmul(edge_attr, wi2, 8000)
    h = _sc_h0(q, r, srcA3)
    p = _tc_p(h, W_m)
    for _ in range(DEPTH):
        agg = _sc_spmm(p, src4, dst4)
        h, p = _tc_prop(h, agg, W_m)
    nacc = _sc_node(q, r, h, src4, dst4)
    return _tc_readout(xp, nacc, batch2d, wa1, wa2,
                       ffn_w1, fb1, ffn_w2, fb2, last_w, lb)
